# Initial kernel scaffold; baseline (speedup 1.0000x reference)
import jax
import jax.numpy as jnp
from jax.experimental import pallas as pl


def kernel(x, edge_index, edge_attr, W_in, b_in, W1, b1, g1, be1, W2, b2, g2, be2, W3, b3, g3, be3, Wf, bf, gf, bef, gates):
    raise NotImplementedError("write your pallas kernel here")


# speedup vs baseline: 2.1839x; 2.1839x over previous
import functools

import jax
import jax.numpy as jnp
from jax import lax
from jax.experimental import pallas as pl
from jax.experimental.pallas import tpu as pltpu
from jax.experimental.pallas import tpu_sc as plsc

N = 10000
E = 320000
H = 128

NW = 32
NPT = 320
JUNK = NPT
C = 128
EPAD = 256

_f32 = jnp.float32
_i32 = jnp.int32


def _make_segmax(with_mask: bool):
  mesh = plsc.VectorSubcoreMesh(core_axis_name="c", subcore_axis_name="s")
  out_type = [jax.ShapeDtypeStruct((N, H), _f32)]
  if with_mask:
    out_type.append(jax.ShapeDtypeStruct((NW * NPT,), _f32))

  scratch = [
      pltpu.VMEM((C,), _i32),
      pltpu.VMEM((C,), _i32),
      pltpu.VMEM((C, H), _f32),
      pltpu.VMEM((NPT + 1, H), _f32),
      pltpu.VMEM((128,), _i32),
      pltpu.SemaphoreType.DMA,
  ]
  if with_mask:
    scratch.append(pltpu.VMEM((C,), _f32))
    scratch.append(pltpu.VMEM((NPT + 16,), _f32))

  def body(*refs):
    if with_mask:
      (table, srcs, dsts, params_hbm, zrows, wts,
       out, flags_out,
       sidx, didx, rows, acc, params, sem, wchunk, flg) = refs
    else:
      (table, srcs, dsts, params_hbm, zrows,
       out,
       sidx, didx, rows, acc, params, sem) = refs

    wid = lax.axis_index("s") * 2 + lax.axis_index("c")
    base = pl.multiple_of(wid * NPT, 8)
    owned = jnp.minimum(NPT, N - base)

    pltpu.sync_copy(zrows, acc)
    if with_mask:
      zv = jnp.zeros((16,), _f32)
      for i in range((NPT + 16) // 16):
        flg[pl.ds(i * 16, 16)] = zv

    pltpu.sync_copy(params_hbm, params)
    astart = params[pl.ds(wid, 16)][0]
    nchunks = params[pl.ds(64 + wid, 16)][0]

    lane0 = lax.iota(_i32, 16) == 0

    def chunk_body(k, carry):
      e0 = pl.multiple_of(astart + k * C, 8)
      pltpu.sync_copy(srcs.at[pl.ds(e0, C)], sidx)
      pltpu.sync_copy(dsts.at[pl.ds(e0, C)], didx)
      if with_mask:
        pltpu.sync_copy(wts.at[pl.ds(e0, C)], wchunk)
      pltpu.async_copy(table.at[sidx], rows, sem).wait()

      def group_body(g, carry2):
        g16 = g * 16
        dvec = didx[pl.ds(g16, 16)] - base
        inr = (dvec >= 0) & (dvec < owned)
        dj = jnp.where(inr, dvec, JUNK)
        if with_mask:
          wv = wchunk[pl.ds(g16, 16)]
        for j in range(16):
          d = dj[j]
          if with_mask:
            w = wv[j]
            fv = flg[pl.ds(d, 16)]
            flg[pl.ds(d, 16)] = jnp.where(lane0, jnp.maximum(fv, w), fv)
          for k2 in range(8):
            r = rows[g16 + j, pl.ds(k2 * 16, 16)]
            if with_mask:
              r = r * w
            a = acc[d, pl.ds(k2 * 16, 16)]
            acc[d, pl.ds(k2 * 16, 16)] = jnp.maximum(a, r)
        return carry2

      lax.fori_loop(0, C // 16, group_body, 0)
      return carry

    lax.fori_loop(0, nchunks, chunk_body, 0)

    @pl.when(wid < NW - 1)
    def _():
      pltpu.sync_copy(acc.at[pl.ds(0, NPT)], out.at[pl.ds(base, NPT)])
      if with_mask:
        pltpu.sync_copy(flg.at[pl.ds(0, NPT)],
                        flags_out.at[pl.ds(base, NPT)])

    last = N - (NW - 1) * NPT

    @pl.when(wid == NW - 1)
    def _():
      pltpu.sync_copy(acc.at[pl.ds(0, last)],
                      out.at[pl.ds((NW - 1) * NPT, last)])
      if with_mask:
        pltpu.sync_copy(flg.at[pl.ds(0, last)],
                        flags_out.at[pl.ds((NW - 1) * NPT, last)])

  return pl.kernel(body, out_type=out_type, mesh=mesh,
                   scratch_types=scratch)


BN = 1000
GRID = N // BN


def _lin_relu_body(x_ref, w_ref, b_ref, o_ref):
  o_ref[...] = jnp.maximum(
      jnp.dot(x_ref[...], w_ref[...], preferred_element_type=_f32)
      + b_ref[...], 0.0)


def _lin_relu(x, w, b):
  return pl.pallas_call(
      _lin_relu_body,
      grid=(GRID,),
      in_specs=[
          pl.BlockSpec((BN, H), lambda i: (i, 0)),
          pl.BlockSpec((H, H), lambda i: (0, 0)),
          pl.BlockSpec((1, H), lambda i: (0, 0)),
      ],
      out_specs=pl.BlockSpec((BN, H), lambda i: (i, 0)),
      out_shape=jax.ShapeDtypeStruct((N, H), _f32),
  )(x, w, b.reshape(1, H))


def _layer_body(agg_ref, h_ref, w_ref, b_ref, g_ref, be_ref, gate_ref,
                f_ref, hn_ref, fo_ref):
  out = (jnp.dot(agg_ref[...], w_ref[...], preferred_element_type=_f32)
         + b_ref[...])
  mu = jnp.mean(out, axis=-1, keepdims=True)
  var = jnp.mean((out - mu) * (out - mu), axis=-1, keepdims=True)
  out = (out - mu) / jnp.sqrt(var + 1e-5) * g_ref[...] + be_ref[...]
  hn = jnp.maximum(out + h_ref[...], 0.0)
  hn_ref[...] = hn
  sig = 1.0 / (1.0 + jnp.exp(-gate_ref[...]))
  fo_ref[...] = f_ref[...] + hn * sig


def _layer_tc(agg, h, w, b, g, be, gate, fused):
  return pl.pallas_call(
      _layer_body,
      grid=(GRID,),
      in_specs=[
          pl.BlockSpec((BN, H), lambda i: (i, 0)),
          pl.BlockSpec((BN, H), lambda i: (i, 0)),
          pl.BlockSpec((H, H), lambda i: (0, 0)),
          pl.BlockSpec((1, H), lambda i: (0, 0)),
          pl.BlockSpec((1, H), lambda i: (0, 0)),
          pl.BlockSpec((1, H), lambda i: (0, 0)),
          pl.BlockSpec((1, H), lambda i: (0, 0)),
          pl.BlockSpec((BN, H), lambda i: (i, 0)),
      ],
      out_specs=[
          pl.BlockSpec((BN, H), lambda i: (i, 0)),
          pl.BlockSpec((BN, H), lambda i: (i, 0)),
      ],
      out_shape=[
          jax.ShapeDtypeStruct((N, H), _f32),
          jax.ShapeDtypeStruct((N, H), _f32),
      ],
  )(agg, h, w, b.reshape(1, H), g.reshape(1, H), be.reshape(1, H),
    gate.reshape(1, H), fused)


def _final_body(agg_ref, nf_ref, x0_ref, flg_ref, w_ref, b_ref, g_ref,
                be_ref, res_ref):
  i = pl.program_id(0)

  out = (jnp.dot(agg_ref[...], w_ref[...], preferred_element_type=_f32)
         + b_ref[...])
  mu = jnp.mean(out, axis=-1, keepdims=True)
  var = jnp.mean((out - mu) * (out - mu), axis=-1, keepdims=True)
  out = (out - mu) / jnp.sqrt(var + 1e-5) * g_ref[...] + be_ref[...]
  subf = jnp.maximum(out + nf_ref[...], 0.0)

  neg = jnp.float32(-jnp.inf)
  sink = flg_ref[...] > 0.0
  v2b = jnp.max(jnp.where(sink, subf, neg), axis=0, keepdims=True)
  rem = jnp.logical_not(x0_ref[...] > 0.1)
  v3b = jnp.max(jnp.where(rem, nf_ref[...], neg), axis=0, keepdims=True)
  both = jnp.concatenate([v2b, v3b], axis=0)

  @pl.when(i == 0)
  def _():
    res_ref[...] = jnp.full((2, H), neg, _f32)

  acc = jnp.maximum(res_ref[...], both)
  res_ref[...] = acc

  @pl.when(i == GRID - 1)
  def _():
    res_ref[...] = jnp.where(jnp.isneginf(acc), jnp.float32(1e-4), acc)


def _final_tc(agg, nf, x0, flags, wf, bf, gf, bef):
  return pl.pallas_call(
      _final_body,
      grid=(GRID,),
      in_specs=[
          pl.BlockSpec((BN, H), lambda i: (i, 0)),
          pl.BlockSpec((BN, H), lambda i: (i, 0)),
          pl.BlockSpec((BN, 1), lambda i: (i, 0)),
          pl.BlockSpec((BN, 1), lambda i: (i, 0)),
          pl.BlockSpec((H, H), lambda i: (0, 0)),
          pl.BlockSpec((1, H), lambda i: (0, 0)),
          pl.BlockSpec((1, H), lambda i: (0, 0)),
          pl.BlockSpec((1, H), lambda i: (0, 0)),
      ],
      out_specs=pl.BlockSpec((2, H), lambda i: (0, 0)),
      out_shape=jax.ShapeDtypeStruct((2, H), _f32),
  )(agg, nf, x0, flags, wf, bf.reshape(1, H), gf.reshape(1, H),
    bef.reshape(1, H))


def kernel(x, edge_index, edge_attr, W_in, b_in, W1, b1, g1, be1, W2, b2,
           g2, be2, W3, b3, g3, be3, Wf, bf, gf, bef, gates):
  src = edge_index[0]
  dst = edge_index[1]

  order = jnp.argsort(dst)
  srcs = jnp.concatenate(
      [src[order].astype(_i32), jnp.zeros((EPAD,), _i32)])
  dsts_core = dst[order].astype(_i32)
  dsts = jnp.concatenate([dsts_core, jnp.full((EPAD,), N, _i32)])
  wts = jnp.concatenate(
      [(edge_attr == 3).astype(_f32)[order], jnp.zeros((EPAD,), _f32)])

  tb = jnp.arange(NW, dtype=_i32) * NPT
  starts = jnp.searchsorted(dsts_core, tb).astype(_i32)
  ends = jnp.searchsorted(dsts_core, tb + NPT).astype(_i32)
  astart = (starts // 8) * 8
  nch = (ends - astart + C - 1) // C
  params = jnp.zeros((128,), _i32)
  params = params.at[0:NW].set(astart).at[64:64 + NW].set(nch)

  zrows = jnp.zeros((NPT + 1, H), _f32)

  segmax = _make_segmax(False)
  segmax_masked = _make_segmax(True)

  h = _lin_relu(x, W_in, b_in)
  fused = jnp.zeros((N, H), _f32)
  for (w, b, g, be, gate) in (
      (W1, b1, g1, be1, gates[0]),
      (W2, b2, g2, be2, gates[1]),
      (W3, b3, g3, be3, gates[2]),
  ):
    [agg] = segmax(h, srcs, dsts, params, zrows)
    h, fused = _layer_tc(agg, h, w, b, g, be, gate, fused)

  aggf, flags = segmax_masked(fused, srcs, dsts, params, zrows, wts)

  res = _final_tc(aggf, fused, x[:, 0:1], flags[:N].reshape(N, 1),
                  Wf, bf, gf, bef)
  return res
